# baseline (speedup 1.0000x reference)
import functools

import jax
import jax.numpy as jnp
from jax import lax
from jax.experimental import pallas as pl
from jax.experimental.pallas import tpu as pltpu
from jax.experimental.pallas import tpu_sc as plsc

EMB = 128
BATCH = 16384

NUM_CORES = 2
NUM_SUBCORES = 16
NW = NUM_CORES * NUM_SUBCORES
CHUNK = 128
TOT_CHUNKS = BATCH // CHUNK

CHUNK_ROWS = (4096, 4096, 8192)

TB = 4096


def _gather_body(chunk0, cpw, table_hbm, idx_hbm, out_hbm, idx_v, rows_v,
                 gsem, ssem):
    wid = lax.axis_index("s") * NUM_CORES + lax.axis_index("c")
    row0 = chunk0 + wid * cpw
    pltpu.sync_copy(idx_hbm.at[pl.ds(row0, cpw)], idx_v)
    gets = [
        pltpu.async_copy(table_hbm.at[idx_v.at[j, 0]], rows_v.at[j], gsem)
        for j in range(cpw)
    ]
    puts = []
    for j in range(cpw):
        gets[j].wait()
        puts.append(
            pltpu.async_copy(
                rows_v.at[j],
                out_hbm.at[pl.ds((wid * cpw + j) * CHUNK, CHUNK)], ssem
            )
        )
    for p in puts:
        p.wait()


def _sc_gather(biz_table, idx2, chunk0_rows, nrows):
    cpw = nrows // CHUNK // NW
    mesh = plsc.VectorSubcoreMesh(
        core_axis_name="c", subcore_axis_name="s",
        num_cores=NUM_CORES, num_subcores=NUM_SUBCORES,
    )
    fn = pl.kernel(
        functools.partial(_gather_body, chunk0_rows // CHUNK, cpw),
        out_type=jax.ShapeDtypeStruct((nrows, EMB), jnp.float32),
        mesh=mesh,
        scratch_types=[
            pltpu.VMEM((cpw, 1, CHUNK), jnp.int32),
            pltpu.VMEM((cpw, CHUNK, EMB), jnp.float32),
            pltpu.SemaphoreType.DMA,
            pltpu.SemaphoreType.DMA,
        ],
    )
    return fn(biz_table, idx2)


def _mlp_body(users_ref, ue_ref, be_ref, W1_ref, b1_ref, W2_ref, b2_ref,
              W3_ref, b3_ref, W4_ref, b4_ref, out_ref):
    ue = ue_ref[0]
    u1 = jnp.dot(ue, W1_ref[:EMB, :], preferred_element_type=jnp.float32)
    x = be_ref[...]
    h = jnp.dot(x, W1_ref[EMB:, :], preferred_element_type=jnp.float32)
    h = jnp.maximum(h + (u1 + b1_ref[...]), 0.0)
    h = jnp.dot(h, W2_ref[...], preferred_element_type=jnp.float32)
    h = jnp.maximum(h + b2_ref[...], 0.0)
    h = jnp.dot(h, W3_ref[...], preferred_element_type=jnp.float32)
    h = jnp.maximum(h + b3_ref[...], 0.0)
    o = lax.dot_general(W4_ref[...], h, (((1,), (1,)), ((), ())),
                        preferred_element_type=jnp.float32)
    out_ref[...] = o + b4_ref[...]


def _tc_mlp(users, be, user_table, W1, b1, W2, b2, W3, b3, W4t, b4):
    ut3 = user_table.reshape(user_table.shape[0], 1, EMB)
    nb = be.shape[0]
    grid = (nb // TB,)
    fn = pl.pallas_call(
        _mlp_body,
        grid_spec=pltpu.PrefetchScalarGridSpec(
            num_scalar_prefetch=1,
            grid=grid,
            in_specs=[
                pl.BlockSpec((1, 1, EMB), lambda i, u: (u[0], 0, 0)),
                pl.BlockSpec((TB, EMB), lambda i, u: (i, 0)),
                pl.BlockSpec((2 * EMB, 1024), lambda i, u: (0, 0)),
                pl.BlockSpec((1, 1024), lambda i, u: (0, 0)),
                pl.BlockSpec((1024, 512), lambda i, u: (0, 0)),
                pl.BlockSpec((1, 512), lambda i, u: (0, 0)),
                pl.BlockSpec((512, 256), lambda i, u: (0, 0)),
                pl.BlockSpec((1, 256), lambda i, u: (0, 0)),
                pl.BlockSpec((1, 256), lambda i, u: (0, 0)),
                pl.BlockSpec((1, 1), lambda i, u: (0, 0)),
            ],
            out_specs=pl.BlockSpec((1, TB), lambda i, u: (0, i)),
        ),
        out_shape=jax.ShapeDtypeStruct((1, nb), jnp.float32),
    )
    return fn(users, ut3, be, W1, b1.reshape(1, -1), W2, b2.reshape(1, -1),
              W3, b3.reshape(1, -1), W4t, b4.reshape(1, -1))


def kernel(users, businesses, user_table, biz_table, W1, b1, W2, b2, W3, b3,
           W4, b4):
    idx2 = businesses.reshape(TOT_CHUNKS, 1, CHUNK)
    W4t = W4.reshape(1, -1)
    bes, base = [], 0
    for nrows in CHUNK_ROWS:
        bes.append(_sc_gather(biz_table, idx2, base, nrows))
        base += nrows
    outs = [
        _tc_mlp(users, be, user_table, W1, b1, W2, b2, W3, b3, W4t, b4)
        for be in bes
    ]
    return jnp.concatenate(outs, axis=1).reshape(BATCH)


# speedup vs baseline: 1.0461x; 1.0461x over previous
import functools

import jax
import jax.numpy as jnp
from jax import lax
from jax.experimental import pallas as pl
from jax.experimental.pallas import tpu as pltpu
from jax.experimental.pallas import tpu_sc as plsc

EMB = 128
BATCH = 16384

NUM_CORES = 2
NUM_SUBCORES = 16
NW = NUM_CORES * NUM_SUBCORES
CHUNK = 128
TOT_CHUNKS = BATCH // CHUNK

CHUNK_ROWS = (8192, 8192)

TB = 4096


def _gather_body(chunk0, cpw, table_hbm, idx_hbm, out_hbm, idx_v, rows_v,
                 gsem, ssem):
    wid = lax.axis_index("s") * NUM_CORES + lax.axis_index("c")
    row0 = chunk0 + wid * cpw
    pltpu.sync_copy(idx_hbm.at[pl.ds(row0, cpw)], idx_v)
    gets = [
        pltpu.async_copy(table_hbm.at[idx_v.at[j, 0]], rows_v.at[j], gsem)
        for j in range(cpw)
    ]
    puts = []
    for j in range(cpw):
        gets[j].wait()
        puts.append(
            pltpu.async_copy(
                rows_v.at[j],
                out_hbm.at[pl.ds((wid * cpw + j) * CHUNK, CHUNK)], ssem
            )
        )
    for p in puts:
        p.wait()


def _sc_gather(biz_table, idx2, chunk0_rows, nrows):
    cpw = nrows // CHUNK // NW
    mesh = plsc.VectorSubcoreMesh(
        core_axis_name="c", subcore_axis_name="s",
        num_cores=NUM_CORES, num_subcores=NUM_SUBCORES,
    )
    fn = pl.kernel(
        functools.partial(_gather_body, chunk0_rows // CHUNK, cpw),
        out_type=jax.ShapeDtypeStruct((nrows, EMB), jnp.float32),
        mesh=mesh,
        scratch_types=[
            pltpu.VMEM((cpw, 1, CHUNK), jnp.int32),
            pltpu.VMEM((cpw, CHUNK, EMB), jnp.float32),
            pltpu.SemaphoreType.DMA,
            pltpu.SemaphoreType.DMA,
        ],
    )
    return fn(biz_table, idx2)


def _mlp_body(users_ref, ue_ref, be_ref, W1_ref, b1_ref, W2_ref, b2_ref,
              W3_ref, b3_ref, W4_ref, b4_ref, out_ref):
    ue = ue_ref[0]
    u1 = jnp.dot(ue, W1_ref[:EMB, :], preferred_element_type=jnp.float32)
    x = be_ref[...]
    h = jnp.dot(x, W1_ref[EMB:, :], preferred_element_type=jnp.float32)
    h = jnp.maximum(h + (u1 + b1_ref[...]), 0.0)
    h = jnp.dot(h, W2_ref[...], preferred_element_type=jnp.float32)
    h = jnp.maximum(h + b2_ref[...], 0.0)
    h = jnp.dot(h, W3_ref[...], preferred_element_type=jnp.float32)
    h = jnp.maximum(h + b3_ref[...], 0.0)
    o = lax.dot_general(W4_ref[...], h, (((1,), (1,)), ((), ())),
                        preferred_element_type=jnp.float32)
    out_ref[...] = o + b4_ref[...]


def _tc_mlp(users, be, user_table, W1, b1, W2, b2, W3, b3, W4t, b4):
    ut3 = user_table.reshape(user_table.shape[0], 1, EMB)
    nb = be.shape[0]
    grid = (nb // TB,)
    fn = pl.pallas_call(
        _mlp_body,
        grid_spec=pltpu.PrefetchScalarGridSpec(
            num_scalar_prefetch=1,
            grid=grid,
            in_specs=[
                pl.BlockSpec((1, 1, EMB), lambda i, u: (u[0], 0, 0)),
                pl.BlockSpec((TB, EMB), lambda i, u: (i, 0)),
                pl.BlockSpec((2 * EMB, 1024), lambda i, u: (0, 0)),
                pl.BlockSpec((1, 1024), lambda i, u: (0, 0)),
                pl.BlockSpec((1024, 512), lambda i, u: (0, 0)),
                pl.BlockSpec((1, 512), lambda i, u: (0, 0)),
                pl.BlockSpec((512, 256), lambda i, u: (0, 0)),
                pl.BlockSpec((1, 256), lambda i, u: (0, 0)),
                pl.BlockSpec((1, 256), lambda i, u: (0, 0)),
                pl.BlockSpec((1, 1), lambda i, u: (0, 0)),
            ],
            out_specs=pl.BlockSpec((1, TB), lambda i, u: (0, i)),
        ),
        out_shape=jax.ShapeDtypeStruct((1, nb), jnp.float32),
    )
    return fn(users, ut3, be, W1, b1.reshape(1, -1), W2, b2.reshape(1, -1),
              W3, b3.reshape(1, -1), W4t, b4.reshape(1, -1))


def kernel(users, businesses, user_table, biz_table, W1, b1, W2, b2, W3, b3,
           W4, b4):
    idx2 = businesses.reshape(TOT_CHUNKS, 1, CHUNK)
    W4t = W4.reshape(1, -1)
    bes, base = [], 0
    for nrows in CHUNK_ROWS:
        bes.append(_sc_gather(biz_table, idx2, base, nrows))
        base += nrows
    outs = [
        _tc_mlp(users, be, user_table, W1, b1, W2, b2, W3, b3, W4t, b4)
        for be in bes
    ]
    return jnp.concatenate(outs, axis=0).reshape(BATCH)
